# baseline (speedup 1.0000x reference)
import jax
import jax.numpy as jnp
from jax.experimental import pallas as pl
from jax.experimental.pallas import tpu as pltpu

_CH = 256
_NBUF = 8
_FIXROWS = 16
_THRESHOLD = 0.5


def _body(img_hbm, out_hbm, bufs, fixbufs, in_sems, out_sems, fin_sems, fout_sems):
    B, C, H, W = img_hbm.shape
    n_per_img = H // _CH
    n_chunks = B * n_per_img

    def chunk_in(k):
        b, j = divmod(k, n_per_img)
        slot = k % _NBUF
        return pltpu.make_async_copy(
            img_hbm.at[b, :, pl.ds(j * _CH, _CH), :], bufs.at[slot], in_sems.at[slot]
        )

    def chunk_out(k):
        b, j = divmod(k, n_per_img)
        slot = k % _NBUF
        return pltpu.make_async_copy(
            bufs.at[slot], out_hbm.at[b, :, pl.ds(j * _CH, _CH), :], out_sems.at[slot]
        )

    def fix_window(scal):
        return pl.multiple_of(
            jnp.clip(((scal[0] - 2) // 8) * 8, 0, H - _FIXROWS), 8
        )

    def fix_in(b, scal):
        ys = fix_window(scal)
        return pltpu.make_async_copy(
            img_hbm.at[b, :, pl.ds(ys, _FIXROWS), :],
            fixbufs.at[b % 2],
            fin_sems.at[b % 2],
        )

    def fix_out(b, scal):
        ys = fix_window(scal)
        return pltpu.make_async_copy(
            fixbufs.at[b % 2],
            out_hbm.at[b, :, pl.ds(ys, _FIXROWS), :],
            fout_sems.at[b % 2],
        )

    def mask_fixbuf(b, scal):
        y1, nx0, x1 = scal
        x0 = -nx0
        y_lo = jnp.maximum(y1 - 3, 0)
        ys = fix_window(scal)
        rows = jax.lax.broadcasted_iota(jnp.int32, (_FIXROWS, W), 0) + ys
        cols = jax.lax.broadcasted_iota(jnp.int32, (_FIXROWS, W), 1)
        m = (rows <= y1) & (rows > y_lo) & (cols >= x0) & (cols < x1)
        fixbufs[b % 2] = jnp.where(m[None], 0.0, fixbufs[b % 2])

    def reduce_chunk(k):
        b, j = divmod(k, n_per_img)
        slot = k % _NBUF
        gray = (
            0.2989 * bufs[slot, 0]
            + 0.587 * bufs[slot, 1]
            + 0.114 * bufs[slot, 2]
        )
        black01 = jnp.where(gray < _THRESHOLD, 1.0, 0.0)
        rowany = jnp.max(black01, axis=1, keepdims=True)
        colany = jnp.max(black01, axis=0, keepdims=True)
        rowsi = jax.lax.broadcasted_iota(jnp.int32, (_CH, 1), 0) + j * _CH
        colsi = jax.lax.broadcasted_iota(jnp.int32, (1, W), 1)
        y1 = jnp.max(jnp.where(rowany > 0.5, rowsi, -1))
        nx0 = jnp.max(jnp.where(colany > 0.5, -colsi, -W))
        x1 = jnp.max(jnp.where(colany > 0.5, colsi, -1))
        return y1, nx0, x1

    out_waited = [False] * n_chunks

    def wait_out(k):
        if not out_waited[k]:
            chunk_out(k).wait()
            out_waited[k] = True

    scals = {}
    partial = None

    for k in range(min(_NBUF, n_chunks)):
        chunk_in(k).start()

    for k in range(n_chunks):
        b, j = divmod(k, n_per_img)
        kp = k + _NBUF // 2
        if _NBUF <= kp < n_chunks:
            wait_out(kp - _NBUF)
            chunk_in(kp).start()

        chunk_in(k).wait()
        chunk_out(k).start()
        t = reduce_chunk(k)
        partial = t if j == 0 else tuple(map(jnp.maximum, partial, t))

        if j == n_per_img - 1:
            scals[b] = partial
            if b >= 2:
                fix_out(b - 2, scals[b - 2]).wait()
            fix_in(b, scals[b]).start()
            if b >= 1:
                fix_in(b - 1, scals[b - 1]).wait()
                mask_fixbuf(b - 1, scals[b - 1])
                for kk in range((b - 1) * n_per_img, b * n_per_img):
                    wait_out(kk)
                fix_out(b - 1, scals[b - 1]).start()

    for k in range(n_chunks):
        wait_out(k)
    fix_in(B - 1, scals[B - 1]).wait()
    mask_fixbuf(B - 1, scals[B - 1])
    fix_out(B - 1, scals[B - 1]).start()
    fix_out(B - 2, scals[B - 2]).wait()
    fix_out(B - 1, scals[B - 1]).wait()


def kernel(img_tensor):
    B, C, H, W = img_tensor.shape
    return pl.pallas_call(
        _body,
        in_specs=[pl.BlockSpec(memory_space=pl.ANY)],
        out_specs=pl.BlockSpec(memory_space=pl.ANY),
        out_shape=jax.ShapeDtypeStruct((B, C, H, W), jnp.float32),
        scratch_shapes=[
            pltpu.VMEM((_NBUF, C, _CH, W), jnp.float32),
            pltpu.VMEM((2, C, _FIXROWS, W), jnp.float32),
            pltpu.SemaphoreType.DMA((_NBUF,)),
            pltpu.SemaphoreType.DMA((_NBUF,)),
            pltpu.SemaphoreType.DMA((2,)),
            pltpu.SemaphoreType.DMA((2,)),
        ],
    )(img_tensor)


# speedup vs baseline: 1.0174x; 1.0174x over previous
import jax
import jax.numpy as jnp
from jax.experimental import pallas as pl
from jax.experimental.pallas import tpu as pltpu

_CH = 128
_NBUF = 16
_FIXROWS = 16
_THRESHOLD = 0.5


def _body(img_hbm, out_hbm, bufs, fixbufs, in_sems, out_sems, fin_sems, fout_sems):
    B, C, H, W = img_hbm.shape
    n_per_img = H // _CH
    n_chunks = B * n_per_img

    def chunk_in(k):
        b, j = divmod(k, n_per_img)
        slot = k % _NBUF
        return pltpu.make_async_copy(
            img_hbm.at[b, :, pl.ds(j * _CH, _CH), :], bufs.at[slot], in_sems.at[slot]
        )

    def chunk_out(k):
        b, j = divmod(k, n_per_img)
        slot = k % _NBUF
        return pltpu.make_async_copy(
            bufs.at[slot], out_hbm.at[b, :, pl.ds(j * _CH, _CH), :], out_sems.at[slot]
        )

    def fix_window(scal):
        return pl.multiple_of(
            jnp.clip(((scal[0] - 2) // 8) * 8, 0, H - _FIXROWS), 8
        )

    def fix_in(b, scal):
        ys = fix_window(scal)
        return pltpu.make_async_copy(
            img_hbm.at[b, :, pl.ds(ys, _FIXROWS), :],
            fixbufs.at[b % 2],
            fin_sems.at[b % 2],
        )

    def fix_out(b, scal):
        ys = fix_window(scal)
        return pltpu.make_async_copy(
            fixbufs.at[b % 2],
            out_hbm.at[b, :, pl.ds(ys, _FIXROWS), :],
            fout_sems.at[b % 2],
        )

    def mask_fixbuf(b, scal):
        y1, nx0, x1 = scal
        x0 = -nx0
        y_lo = jnp.maximum(y1 - 3, 0)
        ys = fix_window(scal)
        rows = jax.lax.broadcasted_iota(jnp.int32, (_FIXROWS, W), 0) + ys
        cols = jax.lax.broadcasted_iota(jnp.int32, (_FIXROWS, W), 1)
        m = (rows <= y1) & (rows > y_lo) & (cols >= x0) & (cols < x1)
        fixbufs[b % 2] = jnp.where(m[None], 0.0, fixbufs[b % 2])

    def reduce_chunk(k):
        b, j = divmod(k, n_per_img)
        slot = k % _NBUF
        gray = (
            0.2989 * bufs[slot, 0]
            + 0.587 * bufs[slot, 1]
            + 0.114 * bufs[slot, 2]
        )
        black01 = jnp.where(gray < _THRESHOLD, 1.0, 0.0)
        rowany = jnp.max(black01, axis=1, keepdims=True)
        colany = jnp.max(black01, axis=0, keepdims=True)
        rowsi = jax.lax.broadcasted_iota(jnp.int32, (_CH, 1), 0) + j * _CH
        colsi = jax.lax.broadcasted_iota(jnp.int32, (1, W), 1)
        y1 = jnp.max(jnp.where(rowany > 0.5, rowsi, -1))
        nx0 = jnp.max(jnp.where(colany > 0.5, -colsi, -W))
        x1 = jnp.max(jnp.where(colany > 0.5, colsi, -1))
        return y1, nx0, x1

    out_waited = [False] * n_chunks

    def wait_out(k):
        if not out_waited[k]:
            chunk_out(k).wait()
            out_waited[k] = True

    scals = {}
    partial = None

    for k in range(min(_NBUF, n_chunks)):
        chunk_in(k).start()

    for k in range(n_chunks):
        b, j = divmod(k, n_per_img)
        kp = k + _NBUF // 2
        if _NBUF <= kp < n_chunks:
            wait_out(kp - _NBUF)
            chunk_in(kp).start()

        chunk_in(k).wait()
        chunk_out(k).start()
        t = reduce_chunk(k)
        partial = t if j == 0 else tuple(map(jnp.maximum, partial, t))

        if j == n_per_img - 1:
            scals[b] = partial
            if b >= 2:
                fix_out(b - 2, scals[b - 2]).wait()
            fix_in(b, scals[b]).start()
            if b >= 1:
                fix_in(b - 1, scals[b - 1]).wait()
                mask_fixbuf(b - 1, scals[b - 1])
                for kk in range((b - 1) * n_per_img, b * n_per_img):
                    wait_out(kk)
                fix_out(b - 1, scals[b - 1]).start()

    for k in range(n_chunks):
        wait_out(k)
    fix_in(B - 1, scals[B - 1]).wait()
    mask_fixbuf(B - 1, scals[B - 1])
    fix_out(B - 1, scals[B - 1]).start()
    fix_out(B - 2, scals[B - 2]).wait()
    fix_out(B - 1, scals[B - 1]).wait()


def kernel(img_tensor):
    B, C, H, W = img_tensor.shape
    return pl.pallas_call(
        _body,
        in_specs=[pl.BlockSpec(memory_space=pl.ANY)],
        out_specs=pl.BlockSpec(memory_space=pl.ANY),
        out_shape=jax.ShapeDtypeStruct((B, C, H, W), jnp.float32),
        scratch_shapes=[
            pltpu.VMEM((_NBUF, C, _CH, W), jnp.float32),
            pltpu.VMEM((2, C, _FIXROWS, W), jnp.float32),
            pltpu.SemaphoreType.DMA((_NBUF,)),
            pltpu.SemaphoreType.DMA((_NBUF,)),
            pltpu.SemaphoreType.DMA((2,)),
            pltpu.SemaphoreType.DMA((2,)),
        ],
    )(img_tensor)
